# baseline (speedup 1.0000x reference)
import functools

import jax
import jax.numpy as jnp
from jax.experimental import pallas as pl
from jax.experimental.pallas import tpu as pltpu

E = 8
TOP_K = 2
D = 1024
I = 512
IS = 1408

TM = 256


def _moe_kernel(x_ref, rw_ref, gu_ref, dn_ref, sg_ref, su_ref, sd_ref, seg_ref,
                out_ref):
    e = pl.program_id(1)
    x = x_ref[...]

    xb = x.astype(jnp.bfloat16)

    @pl.when(e == 0)
    def _init():
        g = jax.lax.dot_general(xb, sg_ref[...].astype(jnp.bfloat16),
                                (((1,), (1,)), ((), ())),
                                preferred_element_type=jnp.float32)
        u = jax.lax.dot_general(xb, su_ref[...].astype(jnp.bfloat16),
                                (((1,), (1,)), ((), ())),
                                preferred_element_type=jnp.float32)
        h = (g * jax.lax.logistic(g)) * u
        shared = jax.lax.dot_general(h.astype(jnp.bfloat16),
                                     sd_ref[...].astype(jnp.bfloat16),
                                     (((1,), (1,)), ((), ())),
                                     preferred_element_type=jnp.float32)
        sgate = jax.lax.logistic(
            jax.lax.dot_general(x, seg_ref[...], (((1,), (1,)), ((), ())),
                                preferred_element_type=jnp.float32))
        out_ref[...] = sgate * shared

    logits = jax.lax.dot_general(x, rw_ref[...], (((1,), (1,)), ((), ())),
                                 preferred_element_type=jnp.float32)
    m1 = jnp.max(logits, axis=-1, keepdims=True)
    i1 = jnp.argmax(logits, axis=-1, keepdims=True)
    eids = jax.lax.broadcasted_iota(jnp.int32, logits.shape, 1)
    masked = jnp.where(eids == i1, -jnp.inf, logits)
    m2 = jnp.max(masked, axis=-1, keepdims=True)
    i2 = jnp.argmax(masked, axis=-1, keepdims=True)
    w1 = jax.lax.logistic(m1 - m2)
    w2 = 1.0 - w1
    we = jnp.where(i1 == e, w1, jnp.where(i2 == e, w2, 0.0))

    gu = jax.lax.dot_general(xb, gu_ref[0].astype(jnp.bfloat16),
                             (((1,), (1,)), ((), ())),
                             preferred_element_type=jnp.float32)
    g = gu[:, :I]
    u = gu[:, I:]
    h = (g * jax.lax.logistic(g)) * u
    o = jax.lax.dot_general(h.astype(jnp.bfloat16),
                            dn_ref[0].astype(jnp.bfloat16),
                            (((1,), (1,)), ((), ())),
                            preferred_element_type=jnp.float32)
    out_ref[...] += we * o


@functools.partial(jax.jit, static_argnames=())
def kernel(hidden_states, router_weight, gate_up_proj, down_proj,
           shared_gate_w, shared_up_w, shared_down_w, shared_expert_gate_w):
    b, s, d = hidden_states.shape
    x = hidden_states.reshape(-1, d)
    t = x.shape[0]
    nt = t // TM

    out = pl.pallas_call(
        _moe_kernel,
        grid=(nt, E),
        in_specs=[
            pl.BlockSpec((TM, D), lambda i, e: (i, 0)),
            pl.BlockSpec((E, D), lambda i, e: (0, 0)),
            pl.BlockSpec((1, 2 * I, D), lambda i, e: (e, 0, 0)),
            pl.BlockSpec((1, D, I), lambda i, e: (e, 0, 0)),
            pl.BlockSpec((IS, D), lambda i, e: (0, 0)),
            pl.BlockSpec((IS, D), lambda i, e: (0, 0)),
            pl.BlockSpec((D, IS), lambda i, e: (0, 0)),
            pl.BlockSpec((1, D), lambda i, e: (0, 0)),
        ],
        out_specs=pl.BlockSpec((TM, D), lambda i, e: (i, 0)),
        out_shape=jax.ShapeDtypeStruct((t, d), jnp.float32),
        compiler_params=pltpu.CompilerParams(
            dimension_semantics=("parallel", "arbitrary"),
        ),
    )(x, router_weight, gate_up_proj, down_proj,
      shared_gate_w, shared_up_w, shared_down_w, shared_expert_gate_w)

    return out.reshape(b, s, d)


# speedup vs baseline: 1.2248x; 1.2248x over previous
import functools

import jax
import jax.numpy as jnp
from jax.experimental import pallas as pl
from jax.experimental.pallas import tpu as pltpu

E = 8
TOP_K = 2
D = 1024
I = 512
IS = 1408

TM = 256


def _shared_kernel(x_ref, rw_ref, sg_ref, su_ref, sd_ref, seg_ref,
                   out_ref, w_ref, sgb_ref, sub_ref, sdb_ref):
    t = pl.program_id(0)

    @pl.when(t == 0)
    def _cast():
        sgb_ref[...] = sg_ref[...].astype(jnp.bfloat16)
        sub_ref[...] = su_ref[...].astype(jnp.bfloat16)
        sdb_ref[...] = sd_ref[...].astype(jnp.bfloat16)

    x = x_ref[...]
    xb = x.astype(jnp.bfloat16)

    logits = jax.lax.dot_general(x, rw_ref[...], (((1,), (1,)), ((), ())),
                                 preferred_element_type=jnp.float32)
    m1 = jnp.max(logits, axis=-1, keepdims=True)
    i1 = jnp.argmax(logits, axis=-1, keepdims=True)
    eids = jax.lax.broadcasted_iota(jnp.int32, logits.shape, 1)
    masked = jnp.where(eids == i1, -jnp.inf, logits)
    m2 = jnp.max(masked, axis=-1, keepdims=True)
    i2 = jnp.argmax(masked, axis=-1, keepdims=True)
    w1 = jax.lax.logistic(m1 - m2)
    w2 = 1.0 - w1
    w_ref[...] = jnp.where(eids == i1, w1, jnp.where(eids == i2, w2, 0.0))

    g = jax.lax.dot_general(xb, sgb_ref[...], (((1,), (1,)), ((), ())),
                            preferred_element_type=jnp.float32)
    u = jax.lax.dot_general(xb, sub_ref[...], (((1,), (1,)), ((), ())),
                            preferred_element_type=jnp.float32)
    h = (g * jax.lax.logistic(g)) * u
    shared = jax.lax.dot_general(h.astype(jnp.bfloat16), sdb_ref[...],
                                 (((1,), (1,)), ((), ())),
                                 preferred_element_type=jnp.float32)
    sgate = jax.lax.logistic(
        jax.lax.dot_general(x, seg_ref[...], (((1,), (1,)), ((), ())),
                            preferred_element_type=jnp.float32))
    out_ref[...] = sgate * shared


def _experts_kernel(x_ref, w_ref, base_ref, gu_ref, dn_ref,
                    out_ref, wgu_ref, wdn_ref):
    e = pl.program_id(0)
    t = pl.program_id(1)

    @pl.when(t == 0)
    def _cast():
        wgu_ref[...] = gu_ref[0].astype(jnp.bfloat16)
        wdn_ref[...] = dn_ref[0].astype(jnp.bfloat16)

    xb = x_ref[...].astype(jnp.bfloat16)
    gu = jax.lax.dot_general(xb, wgu_ref[...], (((1,), (1,)), ((), ())),
                             preferred_element_type=jnp.float32)
    g = gu[:, :I]
    u = gu[:, I:]
    h = (g * jax.lax.logistic(g)) * u
    o = jax.lax.dot_general(h.astype(jnp.bfloat16), wdn_ref[...],
                            (((1,), (1,)), ((), ())),
                            preferred_element_type=jnp.float32)
    wall = w_ref[...]
    eids = jax.lax.broadcasted_iota(jnp.int32, wall.shape, 1)
    we = jnp.sum(jnp.where(eids == e, wall, 0.0), axis=1, keepdims=True)
    row = t * TM

    @pl.when(e == 0)
    def _init():
        out_ref[pl.ds(row, TM), :] = base_ref[...] + we * o

    @pl.when(e != 0)
    def _acc():
        out_ref[pl.ds(row, TM), :] += we * o


@functools.partial(jax.jit, static_argnames=())
def kernel(hidden_states, router_weight, gate_up_proj, down_proj,
           shared_gate_w, shared_up_w, shared_down_w, shared_expert_gate_w):
    b, s, d = hidden_states.shape
    x = hidden_states.reshape(-1, d)
    t = x.shape[0]
    nt = t // TM

    base, w = pl.pallas_call(
        _shared_kernel,
        grid=(nt,),
        in_specs=[
            pl.BlockSpec((TM, D), lambda i: (i, 0)),
            pl.BlockSpec((E, D), lambda i: (0, 0)),
            pl.BlockSpec((IS, D), lambda i: (0, 0)),
            pl.BlockSpec((IS, D), lambda i: (0, 0)),
            pl.BlockSpec((D, IS), lambda i: (0, 0)),
            pl.BlockSpec((1, D), lambda i: (0, 0)),
        ],
        out_specs=[
            pl.BlockSpec((TM, D), lambda i: (i, 0)),
            pl.BlockSpec((TM, E), lambda i: (i, 0)),
        ],
        out_shape=[
            jax.ShapeDtypeStruct((t, d), jnp.float32),
            jax.ShapeDtypeStruct((t, E), jnp.float32),
        ],
        scratch_shapes=[
            pltpu.VMEM((IS, D), jnp.bfloat16),
            pltpu.VMEM((IS, D), jnp.bfloat16),
            pltpu.VMEM((D, IS), jnp.bfloat16),
        ],
        compiler_params=pltpu.CompilerParams(
            dimension_semantics=("arbitrary",),
        ),
    )(x, router_weight, shared_gate_w, shared_up_w, shared_down_w,
      shared_expert_gate_w)

    out = pl.pallas_call(
        _experts_kernel,
        grid=(E, nt),
        in_specs=[
            pl.BlockSpec((TM, D), lambda e, i: (i, 0)),
            pl.BlockSpec((TM, E), lambda e, i: (i, 0)),
            pl.BlockSpec((TM, D), lambda e, i: (i, 0)),
            pl.BlockSpec((1, 2 * I, D), lambda e, i: (e, 0, 0)),
            pl.BlockSpec((1, D, I), lambda e, i: (e, 0, 0)),
        ],
        out_specs=pl.BlockSpec((t, d), lambda e, i: (0, 0)),
        out_shape=jax.ShapeDtypeStruct((t, d), jnp.float32),
        scratch_shapes=[
            pltpu.VMEM((2 * I, D), jnp.bfloat16),
            pltpu.VMEM((D, I), jnp.bfloat16),
        ],
        compiler_params=pltpu.CompilerParams(
            dimension_semantics=("arbitrary", "arbitrary"),
        ),
    )(x, w, base, gate_up_proj, down_proj)

    return out.reshape(b, s, d)


# speedup vs baseline: 1.4212x; 1.1603x over previous
import functools

import jax
import jax.numpy as jnp
from jax.experimental import pallas as pl
from jax.experimental.pallas import tpu as pltpu

E = 8
TOP_K = 2
D = 1024
I = 512
IS = 1408

TM = 512


def _shared_kernel(x_ref, rw_ref, sg_ref, su_ref, sd_ref, seg_ref,
                   out_ref, w_ref, sgb_ref, sub_ref, sdb_ref):
    t = pl.program_id(0)

    @pl.when(t == 0)
    def _cast():
        sgb_ref[...] = sg_ref[...].astype(jnp.bfloat16)
        sub_ref[...] = su_ref[...].astype(jnp.bfloat16)
        sdb_ref[...] = sd_ref[...].astype(jnp.bfloat16)

    x = x_ref[...]
    xb = x.astype(jnp.bfloat16)

    logits = jax.lax.dot_general(x, rw_ref[...], (((1,), (1,)), ((), ())),
                                 preferred_element_type=jnp.float32)
    m1 = jnp.max(logits, axis=-1, keepdims=True)
    i1 = jnp.argmax(logits, axis=-1, keepdims=True)
    eids = jax.lax.broadcasted_iota(jnp.int32, logits.shape, 1)
    masked = jnp.where(eids == i1, -jnp.inf, logits)
    m2 = jnp.max(masked, axis=-1, keepdims=True)
    i2 = jnp.argmax(masked, axis=-1, keepdims=True)
    w1 = jax.lax.logistic(m1 - m2)
    w2 = 1.0 - w1
    w_ref[...] = jnp.where(eids == i1, w1, jnp.where(eids == i2, w2, 0.0))

    g = jax.lax.dot_general(xb, sgb_ref[...], (((1,), (1,)), ((), ())),
                            preferred_element_type=jnp.float32)
    u = jax.lax.dot_general(xb, sub_ref[...], (((1,), (1,)), ((), ())),
                            preferred_element_type=jnp.float32)
    h = (g * jax.lax.logistic(g)) * u
    shared = jax.lax.dot_general(h.astype(jnp.bfloat16), sdb_ref[...],
                                 (((1,), (1,)), ((), ())),
                                 preferred_element_type=jnp.float32)
    sgate = jax.lax.logistic(
        jax.lax.dot_general(x, seg_ref[...], (((1,), (1,)), ((), ())),
                            preferred_element_type=jnp.float32))
    out_ref[...] = sgate * shared


def _experts_kernel(x_ref, w_ref, base_ref, gu_ref, dn_ref,
                    out_ref, wgu_ref, wdn_ref):
    e = pl.program_id(0)
    t = pl.program_id(1)

    @pl.when(t == 0)
    def _cast():
        wgu_ref[...] = gu_ref[0].astype(jnp.bfloat16)
        wdn_ref[...] = dn_ref[0].astype(jnp.bfloat16)

    xb = x_ref[...].astype(jnp.bfloat16)
    gu = jax.lax.dot_general(xb, wgu_ref[...], (((1,), (1,)), ((), ())),
                             preferred_element_type=jnp.float32)
    g = gu[:, :I]
    u = gu[:, I:]
    h = (g * jax.lax.logistic(g)) * u
    o = jax.lax.dot_general(h.astype(jnp.bfloat16), wdn_ref[...],
                            (((1,), (1,)), ((), ())),
                            preferred_element_type=jnp.float32)
    wall = w_ref[...]
    eids = jax.lax.broadcasted_iota(jnp.int32, wall.shape, 1)
    we = jnp.sum(jnp.where(eids == e, wall, 0.0), axis=1, keepdims=True)
    row = t * TM

    @pl.when(e == 0)
    def _init():
        out_ref[pl.ds(row, TM), :] = base_ref[...] + we * o

    @pl.when(e != 0)
    def _acc():
        out_ref[pl.ds(row, TM), :] += we * o


@functools.partial(jax.jit, static_argnames=())
def kernel(hidden_states, router_weight, gate_up_proj, down_proj,
           shared_gate_w, shared_up_w, shared_down_w, shared_expert_gate_w):
    b, s, d = hidden_states.shape
    x = hidden_states.reshape(-1, d)
    t = x.shape[0]
    nt = t // TM

    base, w = pl.pallas_call(
        _shared_kernel,
        grid=(nt,),
        in_specs=[
            pl.BlockSpec((TM, D), lambda i: (i, 0)),
            pl.BlockSpec((E, D), lambda i: (0, 0)),
            pl.BlockSpec((IS, D), lambda i: (0, 0)),
            pl.BlockSpec((IS, D), lambda i: (0, 0)),
            pl.BlockSpec((D, IS), lambda i: (0, 0)),
            pl.BlockSpec((1, D), lambda i: (0, 0)),
        ],
        out_specs=[
            pl.BlockSpec((TM, D), lambda i: (i, 0)),
            pl.BlockSpec((TM, E), lambda i: (i, 0)),
        ],
        out_shape=[
            jax.ShapeDtypeStruct((t, d), jnp.float32),
            jax.ShapeDtypeStruct((t, E), jnp.float32),
        ],
        scratch_shapes=[
            pltpu.VMEM((IS, D), jnp.bfloat16),
            pltpu.VMEM((IS, D), jnp.bfloat16),
            pltpu.VMEM((D, IS), jnp.bfloat16),
        ],
        compiler_params=pltpu.CompilerParams(
            dimension_semantics=("arbitrary",),
        ),
    )(x, router_weight, shared_gate_w, shared_up_w, shared_down_w,
      shared_expert_gate_w)

    out = pl.pallas_call(
        _experts_kernel,
        grid=(E, nt),
        in_specs=[
            pl.BlockSpec((TM, D), lambda e, i: (i, 0)),
            pl.BlockSpec((TM, E), lambda e, i: (i, 0)),
            pl.BlockSpec((TM, D), lambda e, i: (i, 0)),
            pl.BlockSpec((1, 2 * I, D), lambda e, i: (e, 0, 0)),
            pl.BlockSpec((1, D, I), lambda e, i: (e, 0, 0)),
        ],
        out_specs=pl.BlockSpec((t, d), lambda e, i: (0, 0)),
        out_shape=jax.ShapeDtypeStruct((t, d), jnp.float32),
        scratch_shapes=[
            pltpu.VMEM((2 * I, D), jnp.bfloat16),
            pltpu.VMEM((D, I), jnp.bfloat16),
        ],
        compiler_params=pltpu.CompilerParams(
            dimension_semantics=("arbitrary", "arbitrary"),
        ),
    )(x, w, base, gate_up_proj, down_proj)

    return out.reshape(b, s, d)


# speedup vs baseline: 1.4965x; 1.0530x over previous
import functools

import jax
import jax.numpy as jnp
from jax.experimental import pallas as pl
from jax.experimental.pallas import tpu as pltpu

E = 8
TOP_K = 2
D = 1024
I = 512
IS = 1408

TM = 1024


def _shared_kernel(x_ref, rw_ref, sg_ref, su_ref, sd_ref, seg_ref,
                   out_ref, w_ref, sgb_ref, sub_ref, sdb_ref):
    t = pl.program_id(0)

    @pl.when(t == 0)
    def _cast():
        sgb_ref[...] = sg_ref[...].astype(jnp.bfloat16)
        sub_ref[...] = su_ref[...].astype(jnp.bfloat16)
        sdb_ref[...] = sd_ref[...].astype(jnp.bfloat16)

    x = x_ref[...]
    xb = x.astype(jnp.bfloat16)

    logits = jax.lax.dot_general(x, rw_ref[...], (((1,), (1,)), ((), ())),
                                 preferred_element_type=jnp.float32)
    m1 = jnp.max(logits, axis=-1, keepdims=True)
    i1 = jnp.argmax(logits, axis=-1, keepdims=True)
    eids = jax.lax.broadcasted_iota(jnp.int32, logits.shape, 1)
    masked = jnp.where(eids == i1, -jnp.inf, logits)
    m2 = jnp.max(masked, axis=-1, keepdims=True)
    i2 = jnp.argmax(masked, axis=-1, keepdims=True)
    w1 = jax.lax.logistic(m1 - m2)
    w2 = 1.0 - w1
    w_ref[...] = jnp.where(eids == i1, w1, jnp.where(eids == i2, w2, 0.0))

    g = jax.lax.dot_general(xb, sgb_ref[...], (((1,), (1,)), ((), ())),
                            preferred_element_type=jnp.float32)
    u = jax.lax.dot_general(xb, sub_ref[...], (((1,), (1,)), ((), ())),
                            preferred_element_type=jnp.float32)
    h = (g * jax.lax.logistic(g)) * u
    shared = jax.lax.dot_general(h.astype(jnp.bfloat16), sdb_ref[...],
                                 (((1,), (1,)), ((), ())),
                                 preferred_element_type=jnp.float32)
    sgate = jax.lax.logistic(
        jax.lax.dot_general(x, seg_ref[...], (((1,), (1,)), ((), ())),
                            preferred_element_type=jnp.float32))
    out_ref[...] = sgate * shared


def _experts_kernel(x_ref, w_ref, base_ref, gu_ref, dn_ref,
                    out_ref, wgu_ref, wdn_ref):
    e = pl.program_id(0)
    t = pl.program_id(1)

    @pl.when(t == 0)
    def _cast():
        wgu_ref[...] = gu_ref[0].astype(jnp.bfloat16)
        wdn_ref[...] = dn_ref[0].astype(jnp.bfloat16)

    xb = x_ref[...].astype(jnp.bfloat16)
    gu = jax.lax.dot_general(xb, wgu_ref[...], (((1,), (1,)), ((), ())),
                             preferred_element_type=jnp.float32)
    g = gu[:, :I]
    u = gu[:, I:]
    h = (g * jax.lax.logistic(g)) * u
    o = jax.lax.dot_general(h.astype(jnp.bfloat16), wdn_ref[...],
                            (((1,), (1,)), ((), ())),
                            preferred_element_type=jnp.float32)
    wall = w_ref[...]
    eids = jax.lax.broadcasted_iota(jnp.int32, wall.shape, 1)
    we = jnp.sum(jnp.where(eids == e, wall, 0.0), axis=1, keepdims=True)
    row = t * TM

    @pl.when(e == 0)
    def _init():
        out_ref[pl.ds(row, TM), :] = base_ref[...] + we * o

    @pl.when(e != 0)
    def _acc():
        out_ref[pl.ds(row, TM), :] += we * o


@functools.partial(jax.jit, static_argnames=())
def kernel(hidden_states, router_weight, gate_up_proj, down_proj,
           shared_gate_w, shared_up_w, shared_down_w, shared_expert_gate_w):
    b, s, d = hidden_states.shape
    x = hidden_states.reshape(-1, d)
    t = x.shape[0]
    nt = t // TM

    base, w = pl.pallas_call(
        _shared_kernel,
        grid=(nt,),
        in_specs=[
            pl.BlockSpec((TM, D), lambda i: (i, 0)),
            pl.BlockSpec((E, D), lambda i: (0, 0)),
            pl.BlockSpec((IS, D), lambda i: (0, 0)),
            pl.BlockSpec((IS, D), lambda i: (0, 0)),
            pl.BlockSpec((D, IS), lambda i: (0, 0)),
            pl.BlockSpec((1, D), lambda i: (0, 0)),
        ],
        out_specs=[
            pl.BlockSpec((TM, D), lambda i: (i, 0)),
            pl.BlockSpec((TM, E), lambda i: (i, 0)),
        ],
        out_shape=[
            jax.ShapeDtypeStruct((t, d), jnp.float32),
            jax.ShapeDtypeStruct((t, E), jnp.float32),
        ],
        scratch_shapes=[
            pltpu.VMEM((IS, D), jnp.bfloat16),
            pltpu.VMEM((IS, D), jnp.bfloat16),
            pltpu.VMEM((D, IS), jnp.bfloat16),
        ],
        compiler_params=pltpu.CompilerParams(
            dimension_semantics=("arbitrary",),
        ),
    )(x, router_weight, shared_gate_w, shared_up_w, shared_down_w,
      shared_expert_gate_w)

    out = pl.pallas_call(
        _experts_kernel,
        grid=(E, nt),
        in_specs=[
            pl.BlockSpec((TM, D), lambda e, i: (i, 0)),
            pl.BlockSpec((TM, E), lambda e, i: (i, 0)),
            pl.BlockSpec((TM, D), lambda e, i: (i, 0)),
            pl.BlockSpec((1, 2 * I, D), lambda e, i: (e, 0, 0)),
            pl.BlockSpec((1, D, I), lambda e, i: (e, 0, 0)),
        ],
        out_specs=pl.BlockSpec((t, d), lambda e, i: (0, 0)),
        out_shape=jax.ShapeDtypeStruct((t, d), jnp.float32),
        scratch_shapes=[
            pltpu.VMEM((2 * I, D), jnp.bfloat16),
            pltpu.VMEM((D, I), jnp.bfloat16),
        ],
        compiler_params=pltpu.CompilerParams(
            dimension_semantics=("arbitrary", "arbitrary"),
        ),
    )(x, w, base, gate_up_proj, down_proj)

    return out.reshape(b, s, d)


# speedup vs baseline: 1.5821x; 1.0571x over previous
import functools

import jax
import jax.numpy as jnp
from jax.experimental import pallas as pl
from jax.experimental.pallas import tpu as pltpu

E = 8
TOP_K = 2
D = 1024
I = 512
IS = 1408

TM = 512
NT = 2048 // TM


def _moe_kernel(x_ref, rw_ref, gu_ref, dn_ref, sg_ref, su_ref, sd_ref,
                seg_ref, out_ref, w_ref, wgu_ref, wdn_ref):
    e = pl.program_id(0)
    t = pl.program_id(1)
    row = t * TM

    @pl.when(e == 0)
    def _shared_pass():
        x = x_ref[...]
        xb = x.astype(jnp.bfloat16)

        logits = jax.lax.dot_general(x, rw_ref[...], (((1,), (1,)), ((), ())),
                                     preferred_element_type=jnp.float32)
        m1 = jnp.max(logits, axis=-1, keepdims=True)
        i1 = jnp.argmax(logits, axis=-1, keepdims=True)
        eids = jax.lax.broadcasted_iota(jnp.int32, logits.shape, 1)
        masked = jnp.where(eids == i1, -jnp.inf, logits)
        m2 = jnp.max(masked, axis=-1, keepdims=True)
        i2 = jnp.argmax(masked, axis=-1, keepdims=True)
        w1 = jax.lax.logistic(m1 - m2)
        w2 = 1.0 - w1
        w_ref[pl.ds(row, TM), :] = jnp.where(
            eids == i1, w1, jnp.where(eids == i2, w2, 0.0))

        g = jax.lax.dot_general(xb, sg_ref[...].astype(jnp.bfloat16),
                                (((1,), (1,)), ((), ())),
                                preferred_element_type=jnp.float32)
        u = jax.lax.dot_general(xb, su_ref[...].astype(jnp.bfloat16),
                                (((1,), (1,)), ((), ())),
                                preferred_element_type=jnp.float32)
        h = (g * jax.lax.logistic(g)) * u
        shared = jax.lax.dot_general(h.astype(jnp.bfloat16),
                                     sd_ref[...].astype(jnp.bfloat16),
                                     (((1,), (1,)), ((), ())),
                                     preferred_element_type=jnp.float32)
        sgate = jax.lax.logistic(
            jax.lax.dot_general(x, seg_ref[...], (((1,), (1,)), ((), ())),
                                preferred_element_type=jnp.float32))
        out_ref[pl.ds(row, TM), :] = sgate * shared

    @pl.when(e > 0)
    def _expert_pass():
        @pl.when(t == 0)
        def _cast():
            wgu_ref[...] = gu_ref[0].astype(jnp.bfloat16)
            wdn_ref[...] = dn_ref[0].astype(jnp.bfloat16)

        xb = x_ref[...].astype(jnp.bfloat16)
        gu = jax.lax.dot_general(xb, wgu_ref[...], (((1,), (1,)), ((), ())),
                                 preferred_element_type=jnp.float32)
        g = gu[:, :I]
        u = gu[:, I:]
        h = (g * jax.lax.logistic(g)) * u
        o = jax.lax.dot_general(h.astype(jnp.bfloat16), wdn_ref[...],
                                (((1,), (1,)), ((), ())),
                                preferred_element_type=jnp.float32)
        wall = w_ref[pl.ds(row, TM), :]
        eids = jax.lax.broadcasted_iota(jnp.int32, wall.shape, 1)
        we = jnp.sum(jnp.where(eids == e - 1, wall, 0.0), axis=1,
                     keepdims=True)
        out_ref[pl.ds(row, TM), :] += we * o


@functools.partial(jax.jit, static_argnames=())
def kernel(hidden_states, router_weight, gate_up_proj, down_proj,
           shared_gate_w, shared_up_w, shared_down_w, shared_expert_gate_w):
    b, s, d = hidden_states.shape
    x = hidden_states.reshape(-1, d)
    t = x.shape[0]

    out = pl.pallas_call(
        _moe_kernel,
        grid=(E + 1, NT),
        in_specs=[
            pl.BlockSpec((TM, D), lambda e, i: (i, 0)),
            pl.BlockSpec((E, D), lambda e, i: (0, 0)),
            pl.BlockSpec((1, 2 * I, D),
                         lambda e, i: (jnp.maximum(e - 1, 0), 0, 0)),
            pl.BlockSpec((1, D, I),
                         lambda e, i: (jnp.maximum(e - 1, 0), 0, 0)),
            pl.BlockSpec((IS, D), lambda e, i: (0, 0)),
            pl.BlockSpec((IS, D), lambda e, i: (0, 0)),
            pl.BlockSpec((D, IS), lambda e, i: (0, 0)),
            pl.BlockSpec((1, D), lambda e, i: (0, 0)),
        ],
        out_specs=pl.BlockSpec((t, d), lambda e, i: (0, 0)),
        out_shape=jax.ShapeDtypeStruct((t, d), jnp.float32),
        scratch_shapes=[
            pltpu.VMEM((t, E), jnp.float32),
            pltpu.VMEM((2 * I, D), jnp.bfloat16),
            pltpu.VMEM((D, I), jnp.bfloat16),
        ],
        compiler_params=pltpu.CompilerParams(
            dimension_semantics=("arbitrary", "arbitrary"),
        ),
    )(x, router_weight, gate_up_proj, down_proj,
      shared_gate_w, shared_up_w, shared_down_w, shared_expert_gate_w)

    return out.reshape(b, s, d)


# speedup vs baseline: 1.8480x; 1.1681x over previous
import functools

import jax
import jax.numpy as jnp
from jax.experimental import pallas as pl
from jax.experimental.pallas import tpu as pltpu

E = 8
TOP_K = 2
D = 1024
I = 512
IS = 1408

TM = 1024
NT = 2048 // TM


def _moe_kernel(x_ref, rw_ref, gu_ref, dn_ref, sg_ref, su_ref, sd_ref,
                seg_ref, out_ref, w_ref, wgu_ref, wdn_ref):
    e = pl.program_id(0)
    t = pl.program_id(1)
    row = t * TM

    @pl.when(e == 0)
    def _shared_pass():
        x = x_ref[...]
        xb = x.astype(jnp.bfloat16)

        logits = jax.lax.dot_general(x, rw_ref[...], (((1,), (1,)), ((), ())),
                                     preferred_element_type=jnp.float32)
        m1 = jnp.max(logits, axis=-1, keepdims=True)
        i1 = jnp.argmax(logits, axis=-1, keepdims=True)
        eids = jax.lax.broadcasted_iota(jnp.int32, logits.shape, 1)
        masked = jnp.where(eids == i1, -jnp.inf, logits)
        m2 = jnp.max(masked, axis=-1, keepdims=True)
        i2 = jnp.argmax(masked, axis=-1, keepdims=True)
        w1 = jax.lax.logistic(m1 - m2)
        w2 = 1.0 - w1
        w_ref[pl.ds(row, TM), :] = jnp.where(
            eids == i1, w1, jnp.where(eids == i2, w2, 0.0))

        g = jax.lax.dot_general(xb, sg_ref[...].astype(jnp.bfloat16),
                                (((1,), (1,)), ((), ())),
                                preferred_element_type=jnp.float32)
        u = jax.lax.dot_general(xb, su_ref[...].astype(jnp.bfloat16),
                                (((1,), (1,)), ((), ())),
                                preferred_element_type=jnp.float32)
        h = (g * jax.lax.logistic(g)) * u
        shared = jax.lax.dot_general(h.astype(jnp.bfloat16),
                                     sd_ref[...].astype(jnp.bfloat16),
                                     (((1,), (1,)), ((), ())),
                                     preferred_element_type=jnp.float32)
        sgate = jax.lax.logistic(
            jax.lax.dot_general(x, seg_ref[...], (((1,), (1,)), ((), ())),
                                preferred_element_type=jnp.float32))
        out_ref[pl.ds(row, TM), :] = sgate * shared

    @pl.when(e > 0)
    def _expert_pass():
        @pl.when(t == 0)
        def _cast():
            wgu_ref[...] = gu_ref[0].astype(jnp.bfloat16)
            wdn_ref[...] = dn_ref[0].astype(jnp.bfloat16)

        xb = x_ref[...].astype(jnp.bfloat16)
        gu = jax.lax.dot_general(xb, wgu_ref[...], (((1,), (1,)), ((), ())),
                                 preferred_element_type=jnp.float32)
        g = gu[:, :I]
        u = gu[:, I:]
        h = (g * jax.lax.logistic(g)) * u
        o = jax.lax.dot_general(h.astype(jnp.bfloat16), wdn_ref[...],
                                (((1,), (1,)), ((), ())),
                                preferred_element_type=jnp.float32)
        wall = w_ref[pl.ds(row, TM), :]
        eids = jax.lax.broadcasted_iota(jnp.int32, wall.shape, 1)
        we = jnp.sum(jnp.where(eids == e - 1, wall, 0.0), axis=1,
                     keepdims=True)
        out_ref[pl.ds(row, TM), :] += we * o


@functools.partial(jax.jit, static_argnames=())
def kernel(hidden_states, router_weight, gate_up_proj, down_proj,
           shared_gate_w, shared_up_w, shared_down_w, shared_expert_gate_w):
    b, s, d = hidden_states.shape
    x = hidden_states.reshape(-1, d)
    t = x.shape[0]

    out = pl.pallas_call(
        _moe_kernel,
        grid=(E + 1, NT),
        in_specs=[
            pl.BlockSpec((TM, D), lambda e, i: (i, 0)),
            pl.BlockSpec((E, D), lambda e, i: (0, 0)),
            pl.BlockSpec((1, 2 * I, D),
                         lambda e, i: (jnp.maximum(e - 1, 0), 0, 0)),
            pl.BlockSpec((1, D, I),
                         lambda e, i: (jnp.maximum(e - 1, 0), 0, 0)),
            pl.BlockSpec((IS, D), lambda e, i: (0, 0)),
            pl.BlockSpec((IS, D), lambda e, i: (0, 0)),
            pl.BlockSpec((D, IS), lambda e, i: (0, 0)),
            pl.BlockSpec((1, D), lambda e, i: (0, 0)),
        ],
        out_specs=pl.BlockSpec((t, d), lambda e, i: (0, 0)),
        out_shape=jax.ShapeDtypeStruct((t, d), jnp.float32),
        scratch_shapes=[
            pltpu.VMEM((t, E), jnp.float32),
            pltpu.VMEM((2 * I, D), jnp.bfloat16),
            pltpu.VMEM((D, I), jnp.bfloat16),
        ],
        compiler_params=pltpu.CompilerParams(
            dimension_semantics=("arbitrary", "arbitrary"),
        ),
    )(x, router_weight, gate_up_proj, down_proj,
      shared_gate_w, shared_up_w, shared_down_w, shared_expert_gate_w)

    return out.reshape(b, s, d)


# speedup vs baseline: 1.8843x; 1.0196x over previous
import functools

import jax
import jax.numpy as jnp
from jax.experimental import pallas as pl
from jax.experimental.pallas import tpu as pltpu

E = 8
TOP_K = 2
D = 1024
I = 512
IS = 1408

TM = 1024
NT = 2048 // TM


def _moe_kernel(x_ref, rw_ref, gu_ref, dn_ref, sg_ref, su_ref, sd_ref,
                seg_ref, out_ref, w_ref, wgu_ref, wdn_ref):
    e = pl.program_id(0)
    t = pl.program_id(1)
    row = t * TM

    @pl.when(e < E)
    def _cast_next():
        buf = jax.lax.rem(e, 2)
        wgu_ref[buf, pl.ds(t * (2 * I // NT), 2 * I // NT), :] = (
            gu_ref[0, 0].astype(jnp.bfloat16))
        wdn_ref[buf, pl.ds(t * (D // NT), D // NT), :] = (
            dn_ref[0, 0].astype(jnp.bfloat16))

    @pl.when(e == 0)
    def _shared_pass():
        x = x_ref[...]
        xb = x.astype(jnp.bfloat16)

        logits = jax.lax.dot_general(x, rw_ref[...], (((1,), (1,)), ((), ())),
                                     preferred_element_type=jnp.float32)
        m1 = jnp.max(logits, axis=-1, keepdims=True)
        i1 = jnp.argmax(logits, axis=-1, keepdims=True)
        eids = jax.lax.broadcasted_iota(jnp.int32, logits.shape, 1)
        masked = jnp.where(eids == i1, -jnp.inf, logits)
        m2 = jnp.max(masked, axis=-1, keepdims=True)
        i2 = jnp.argmax(masked, axis=-1, keepdims=True)
        w1 = jax.lax.logistic(m1 - m2)
        w2 = 1.0 - w1
        w_ref[pl.ds(row, TM), :] = jnp.where(
            eids == i1, w1, jnp.where(eids == i2, w2, 0.0))

        g = jax.lax.dot_general(xb, sg_ref[...].astype(jnp.bfloat16),
                                (((1,), (1,)), ((), ())),
                                preferred_element_type=jnp.float32)
        u = jax.lax.dot_general(xb, su_ref[...].astype(jnp.bfloat16),
                                (((1,), (1,)), ((), ())),
                                preferred_element_type=jnp.float32)
        h = (g * jax.lax.logistic(g)) * u
        shared = jax.lax.dot_general(h.astype(jnp.bfloat16),
                                     sd_ref[...].astype(jnp.bfloat16),
                                     (((1,), (1,)), ((), ())),
                                     preferred_element_type=jnp.float32)
        sgate = jax.lax.logistic(
            jax.lax.dot_general(x, seg_ref[...], (((1,), (1,)), ((), ())),
                                preferred_element_type=jnp.float32))
        out_ref[pl.ds(row, TM), :] = sgate * shared

    @pl.when(e > 0)
    def _expert_pass():
        buf = jax.lax.rem(e - 1, 2)
        xb = x_ref[...].astype(jnp.bfloat16)
        gu = jax.lax.dot_general(xb, wgu_ref[buf], (((1,), (1,)), ((), ())),
                                 preferred_element_type=jnp.float32)
        g = gu[:, :I]
        u = gu[:, I:]
        h = (g * jax.lax.logistic(g)) * u
        o = jax.lax.dot_general(h.astype(jnp.bfloat16), wdn_ref[buf],
                                (((1,), (1,)), ((), ())),
                                preferred_element_type=jnp.float32)
        wall = w_ref[pl.ds(row, TM), :]
        eids = jax.lax.broadcasted_iota(jnp.int32, wall.shape, 1)
        we = jnp.sum(jnp.where(eids == e - 1, wall, 0.0), axis=1,
                     keepdims=True)
        out_ref[pl.ds(row, TM), :] += we * o


@functools.partial(jax.jit, static_argnames=())
def kernel(hidden_states, router_weight, gate_up_proj, down_proj,
           shared_gate_w, shared_up_w, shared_down_w, shared_expert_gate_w):
    b, s, d = hidden_states.shape
    x = hidden_states.reshape(-1, d)
    t = x.shape[0]

    out = pl.pallas_call(
        _moe_kernel,
        grid=(E + 1, NT),
        in_specs=[
            pl.BlockSpec((TM, D), lambda e, i: (i, 0)),
            pl.BlockSpec((E, D), lambda e, i: (0, 0)),
            pl.BlockSpec((1, 1, 2 * I // NT, D),
                         lambda e, i: (jnp.minimum(e, E - 1),
                                       jnp.where(e < E, i, NT - 1), 0, 0)),
            pl.BlockSpec((1, 1, D // NT, I),
                         lambda e, i: (jnp.minimum(e, E - 1),
                                       jnp.where(e < E, i, NT - 1), 0, 0)),
            pl.BlockSpec((IS, D), lambda e, i: (0, 0)),
            pl.BlockSpec((IS, D), lambda e, i: (0, 0)),
            pl.BlockSpec((D, IS), lambda e, i: (0, 0)),
            pl.BlockSpec((1, D), lambda e, i: (0, 0)),
        ],
        out_specs=pl.BlockSpec((t, d), lambda e, i: (0, 0)),
        out_shape=jax.ShapeDtypeStruct((t, d), jnp.float32),
        scratch_shapes=[
            pltpu.VMEM((t, E), jnp.float32),
            pltpu.VMEM((2, 2 * I, D), jnp.bfloat16),
            pltpu.VMEM((2, D, I), jnp.bfloat16),
        ],
        compiler_params=pltpu.CompilerParams(
            dimension_semantics=("arbitrary", "arbitrary"),
        ),
    )(x, router_weight,
      gate_up_proj.reshape(E, NT, 2 * I // NT, D),
      down_proj.reshape(E, NT, D // NT, I),
      shared_gate_w, shared_up_w, shared_down_w, shared_expert_gate_w)

    return out.reshape(b, s, d)
